# Initial kernel scaffold; baseline (speedup 1.0000x reference)
import math

import jax
import jax.numpy as jnp
from jax.experimental import pallas as pl


_BS = 512


def _pe_add_kernel(x_ref, pos_ref, o_ref):
    j = pl.program_id(1)
    factor = math.sqrt(x_ref.shape[-1])
    pe = pos_ref[pl.ds(j * _BS + 1, _BS), :]
    o_ref[0] = x_ref[0] * factor + pe


def kernel(x, pos_table):
    B, S, D = x.shape
    grid = (B, S // _BS)
    return pl.pallas_call(
        _pe_add_kernel,
        grid=grid,
        in_specs=[
            pl.BlockSpec((1, _BS, D), lambda i, j: (i, j, 0)),
            pl.BlockSpec(pos_table.shape, lambda i, j: (0, 0)),
        ],
        out_specs=pl.BlockSpec((1, _BS, D), lambda i, j: (i, j, 0)),
        out_shape=jax.ShapeDtypeStruct((B, S, D), x.dtype),
    )(x, pos_table)


# speedup vs baseline: 2.2080x; 2.2080x over previous
import math

import jax
import jax.numpy as jnp
from jax.experimental import pallas as pl


_BS = 512


def _pe_add_kernel(x_ref, pos_ref, o_ref):
    j = pl.program_id(1)
    factor = math.sqrt(x_ref.shape[-1])
    win = pos_ref[pl.ds(j * _BS, _BS + 8), :]
    pe = win[1:_BS + 1, :]
    o_ref[0] = x_ref[0] * factor + pe


def kernel(x, pos_table):
    B, S, D = x.shape
    rows_needed = S + 8
    pos_padded = jnp.pad(pos_table, ((0, rows_needed - pos_table.shape[0]), (0, 0)))
    grid = (B, S // _BS)
    return pl.pallas_call(
        _pe_add_kernel,
        grid=grid,
        in_specs=[
            pl.BlockSpec((1, _BS, D), lambda i, j: (i, j, 0)),
            pl.BlockSpec(pos_padded.shape, lambda i, j: (0, 0)),
        ],
        out_specs=pl.BlockSpec((1, _BS, D), lambda i, j: (i, j, 0)),
        out_shape=jax.ShapeDtypeStruct((B, S, D), x.dtype),
    )(x, pos_padded)


# speedup vs baseline: 2.3637x; 1.0705x over previous
import math

import jax
import jax.numpy as jnp
from jax.experimental import pallas as pl
from jax.experimental.pallas import tpu as pltpu


_BS = 1024


def _pe_add_kernel(x_ref, pos_ref, o_ref):
    j = pl.program_id(1)
    factor = math.sqrt(x_ref.shape[-1])
    win = pos_ref[pl.ds(j * _BS, _BS + 8), :]
    pe = win[1:_BS + 1, :]
    o_ref[0] = x_ref[0] * factor + pe


def kernel(x, pos_table):
    B, S, D = x.shape
    rows_needed = S + 8
    pos_padded = jnp.pad(pos_table, ((0, rows_needed - pos_table.shape[0]), (0, 0)))
    grid = (B, S // _BS)
    return pl.pallas_call(
        _pe_add_kernel,
        grid=grid,
        in_specs=[
            pl.BlockSpec((1, _BS, D), lambda i, j: (i, j, 0)),
            pl.BlockSpec(pos_padded.shape, lambda i, j: (0, 0)),
        ],
        out_specs=pl.BlockSpec((1, _BS, D), lambda i, j: (i, j, 0)),
        out_shape=jax.ShapeDtypeStruct((B, S, D), x.dtype),
        compiler_params=pltpu.CompilerParams(
            dimension_semantics=("parallel", "parallel"),
        ),
    )(x, pos_padded)


# speedup vs baseline: 2.4800x; 1.0492x over previous
import math

import jax
import jax.numpy as jnp
from jax.experimental import pallas as pl
from jax.experimental.pallas import tpu as pltpu


_BS = 2048


def _pe_add_kernel(x_ref, pos_ref, o_ref):
    j = pl.program_id(1)
    factor = math.sqrt(x_ref.shape[-1])
    win = pos_ref[pl.ds(j * _BS, _BS + 8), :]
    pe = win[1:_BS + 1, :]
    o_ref[0] = x_ref[0] * factor + pe


def kernel(x, pos_table):
    B, S, D = x.shape
    rows_needed = S + 8
    pos_padded = jnp.pad(pos_table, ((0, rows_needed - pos_table.shape[0]), (0, 0)))
    grid = (B, S // _BS)
    return pl.pallas_call(
        _pe_add_kernel,
        grid=grid,
        in_specs=[
            pl.BlockSpec((1, _BS, D), lambda i, j: (i, j, 0)),
            pl.BlockSpec(pos_padded.shape, lambda i, j: (0, 0)),
        ],
        out_specs=pl.BlockSpec((1, _BS, D), lambda i, j: (i, j, 0)),
        out_shape=jax.ShapeDtypeStruct((B, S, D), x.dtype),
        compiler_params=pltpu.CompilerParams(
            dimension_semantics=("parallel", "parallel"),
        ),
    )(x, pos_padded)


# speedup vs baseline: 3.2998x; 1.3305x over previous
import math

import jax
import jax.numpy as jnp
from jax.experimental import pallas as pl
from jax.experimental.pallas import tpu as pltpu


_CH = 256


def _pe_add_kernel(x_ref, pos_ref, o_ref):
    factor = math.sqrt(x_ref.shape[-1])
    S = x_ref.shape[1]
    n_chunks = S // _CH
    for c in range(n_chunks):
        base = c * _CH
        if base + _CH < S:
            win = pos_ref[pl.ds(base, _CH + 8), :]
            pe = win[1:_CH + 1, :]
        else:
            win = pos_ref[pl.ds(base, _CH), :]
            last = pos_ref[pl.ds(S, 2), :]
            pe = jnp.concatenate([win[1:_CH, :], last[0:1, :]], axis=0)
        o_ref[0, pl.ds(base, _CH), :] = x_ref[0, pl.ds(base, _CH), :] * factor + pe


def kernel(x, pos_table):
    B, S, D = x.shape
    return pl.pallas_call(
        _pe_add_kernel,
        grid=(B,),
        in_specs=[
            pl.BlockSpec((1, S, D), lambda i: (i, 0, 0)),
            pl.BlockSpec(pos_table.shape, lambda i: (0, 0)),
        ],
        out_specs=pl.BlockSpec((1, S, D), lambda i: (i, 0, 0)),
        out_shape=jax.ShapeDtypeStruct((B, S, D), x.dtype),
        compiler_params=pltpu.CompilerParams(
            dimension_semantics=("parallel",),
        ),
    )(x, pos_table)
